# baseline (speedup 1.0000x reference)
import jax
import jax.numpy as jnp
from jax import lax
from jax.experimental import pallas as pl
from jax.experimental.pallas import tpu as pltpu
from jax.experimental.pallas import tpu_sc as plsc

_B = 16384
_D = 64
_SL = 8
_NEG = 5
_L = 16
_NC = 2
_NS = 16
_NW = _NC * _NS
_PER_W = _B // _NW
_CHUNK = 16
_NCHUNK = _PER_W // _CHUNK
_NKC = _NEG * _CHUNK


def _sc_scores_body(gt_hbm, st_hbm, gidx_hbm, sidx_hbm, nidx_hbm,
                    pos_hbm, neg_hbm,
                    gidx_v, sidx_v, nidx_v,
                    g_v, s_v, n_v, pos_v, neg_v, sem):
    cid = lax.axis_index("c")
    sid = lax.axis_index("s")
    wid = sid * _NC + cid
    wbase = wid * _PER_W
    iota = lax.iota(jnp.int32, _L)
    zeros = jnp.zeros((_L,), jnp.int32)
    pltpu.sync_copy(gidx_hbm.at[pl.ds(wbase, _PER_W)], gidx_v)
    pltpu.sync_copy(sidx_hbm.at[pl.ds(wbase, _PER_W)], sidx_v)
    pltpu.sync_copy(nidx_hbm.at[pl.ds(wbase * _NEG, _PER_W * _NEG)], nidx_v)

    def extract(vec, j):
        return jnp.sum(jnp.where(iota == j, vec, 0))

    def chunk_body(c, carry):
        coff = pl.multiple_of(c * _CHUNK, _CHUNK)
        noff = pl.multiple_of(c * _NKC, _L)
        gv = gidx_v[pl.ds(coff, _L)]
        sv = sidx_v[pl.ds(coff, _L)]
        nvs = [nidx_v[pl.ds(noff + t * _L, _L)]
               for t in range(_NEG)]
        def fire_g(j, _):
            rj = extract(gv, j)
            pltpu.async_copy(gt_hbm.at[pl.ds(rj, 1)],
                             g_v.at[pl.ds(j, 1)], sem)
            return 0

        def fire_s(j, _):
            rj = extract(sv, j)
            pltpu.async_copy(st_hbm.at[pl.ds(rj, 1)],
                             s_v.at[pl.ds(j, 1)], sem)
            return 0

        lax.fori_loop(0, _L, fire_g, 0, unroll=4)
        lax.fori_loop(0, _L, fire_s, 0, unroll=4)
        for t in range(_NEG):
            nv = nvs[t]

            def fire_n(j, _, nv=nv, t=t):
                rj = extract(nv, j)
                pltpu.async_copy(st_hbm.at[pl.ds(rj, 1)],
                                 n_v.at[pl.ds(t * _L + j, 1)], sem)
                return 0

            lax.fori_loop(0, _L, fire_n, 0, unroll=4)

        def drain(j, _):
            pltpu.make_async_copy(
                gt_hbm.at[pl.ds(0, 1)],
                g_v.at[pl.ds(0, 1)], sem).wait()
            return 0

        lax.fori_loop(0, (2 + _NEG) * _L, drain, 0, unroll=4)

        def body(d, accs):
            dd = jnp.full((_L,), d, jnp.int32)
            gcol = plsc.load_gather(g_v, [iota, dd])
            scol = plsc.load_gather(s_v, [iota, dd])
            out = [accs[0] + gcol * scol]
            for k in range(_NEG):
                ncol = plsc.load_gather(n_v, [iota * _NEG + k, dd])
                out.append(accs[k + 1] + gcol * ncol)
            return tuple(out)

        z = jnp.zeros((_L,), jnp.float32)
        accs = lax.fori_loop(0, _D, body, (z,) * (1 + _NEG), unroll=4)
        pos_v[pl.ds(coff, _L)] = accs[0]
        for k in range(_NEG):
            neg_v[pl.ds(pl.multiple_of(k * _PER_W + coff, _L), _L)] = (
                accs[k + 1])
        return carry

    lax.fori_loop(0, _NCHUNK, chunk_body, 0)
    pltpu.sync_copy(pos_v, pos_hbm.at[pl.ds(wbase, _PER_W)])
    for k in range(_NEG):
        pltpu.sync_copy(neg_v.at[pl.ds(k * _PER_W, _PER_W)],
                        neg_hbm.at[pl.ds(k * _B + wbase, _PER_W)])


_sc_scores = pl.kernel(
    _sc_scores_body,
    out_type=[jax.ShapeDtypeStruct((_B,), jnp.float32),
              jax.ShapeDtypeStruct((_NEG * _B,), jnp.float32)],
    mesh=plsc.VectorSubcoreMesh(core_axis_name="c", subcore_axis_name="s",
                                num_cores=_NC, num_subcores=_NS),
    scratch_types=[
        pltpu.VMEM((_PER_W,), jnp.int32),
        pltpu.VMEM((_PER_W,), jnp.int32),
        pltpu.VMEM((_PER_W * _NEG,), jnp.int32),
        pltpu.VMEM((_L, _D), jnp.float32),
        pltpu.VMEM((_L, _D), jnp.float32),
        pltpu.VMEM((_NKC, _D), jnp.float32),
        pltpu.VMEM((_PER_W,), jnp.float32),
        pltpu.VMEM((_NEG * _PER_W,), jnp.float32),
        pltpu.SemaphoreType.DMA,
    ],
    compiler_params=pltpu.CompilerParams(needs_layout_passes=False,
                                         use_tc_tiling_on_sc=True),
)


def _tc_loss_body(pos_ref, neg_ref, out_ref):
    p = pos_ref[...]
    pos_loss = -jnp.log(jax.nn.sigmoid(p) + 1e-8)
    acc = jnp.zeros_like(p)
    for k in range(_NEG):
        acc = acc + (-jnp.log(1.0 - jax.nn.sigmoid(neg_ref[k]) + 1e-8))
    out_ref[...] = pos_loss + acc * (1.0 / _NEG)


@jax.jit
def _impl(graph_idx, subgraph_idx, neg_idx, graph_table, subgraph_table):
    nidx_flat = neg_idx.reshape(-1)
    pos, negf = _sc_scores(graph_table, subgraph_table,
                           graph_idx, subgraph_idx, nidx_flat)
    r = _B // 128
    loss = pl.pallas_call(
        _tc_loss_body,
        out_shape=jax.ShapeDtypeStruct((r, 128), jnp.float32),
    )(pos.reshape(r, 128), negf.reshape(_NEG, r, 128))
    return loss.reshape(_B)


def kernel(graph_idx, subgraph_idx, neg_idx, graph_table, subgraph_table):
    return _impl(graph_idx, subgraph_idx, neg_idx, graph_table,
                 subgraph_table)


# speedup vs baseline: 1.4035x; 1.4035x over previous
import jax
import jax.numpy as jnp
from jax import lax
from jax.experimental import pallas as pl
from jax.experimental.pallas import tpu as pltpu
from jax.experimental.pallas import tpu_sc as plsc

_B = 16384
_D = 64
_SL = 8
_NEG = 5
_L = 16
_NC = 2
_NS = 16
_NW = _NC * _NS
_PER_W = _B // _NW
_CHUNK = 16
_NCHUNK = _PER_W // _CHUNK
_NKC = _NEG * _CHUNK


def _sc_scores_body(gt_hbm, st_hbm, gidx_hbm, sidx_hbm, nidx_hbm,
                    pos_hbm, neg_hbm,
                    gidx_v, sidx_v, nidx_v,
                    g_v0, s_v0, n_v0, g_v1, s_v1, n_v1,
                    pos_v, neg_v, sem0, sem1):
    cid = lax.axis_index("c")
    sid = lax.axis_index("s")
    wid = sid * _NC + cid
    wbase = wid * _PER_W
    iota = lax.iota(jnp.int32, _L)
    zeros = jnp.zeros((_L,), jnp.int32)
    bufs = ((g_v0, s_v0, n_v0, sem0), (g_v1, s_v1, n_v1, sem1))
    pltpu.sync_copy(gidx_hbm.at[pl.ds(wbase, _PER_W)], gidx_v)
    pltpu.sync_copy(sidx_hbm.at[pl.ds(wbase, _PER_W)], sidx_v)
    pltpu.sync_copy(nidx_hbm.at[pl.ds(wbase * _NEG, _PER_W * _NEG)], nidx_v)

    def extract(vec, j):
        return jnp.sum(jnp.where(iota == j, vec, 0))

    def fire(c, buf):
        g_v, s_v, n_v, sem = buf
        coff = pl.multiple_of(c * _CHUNK, _CHUNK)
        noff = pl.multiple_of(c * _NKC, _L)
        gv = gidx_v[pl.ds(coff, _L)]
        sv = sidx_v[pl.ds(coff, _L)]
        nvs = [nidx_v[pl.ds(noff + t * _L, _L)] for t in range(_NEG)]
        gb, gs = gv >> 3, gv & (_SL - 1)
        sb, ss = sv >> 3, sv & (_SL - 1)
        nbs = [(nv >> 3, nv & (_SL - 1)) for nv in nvs]

        def fire_g(j, _):
            bj = extract(gb, j)
            sj = extract(gs, j)
            pltpu.async_copy(gt_hbm.at[pl.ds(bj, 1), pl.ds(sj, 1)],
                             g_v.at[pl.ds(j, 1)], sem)
            return 0

        def fire_s(j, _):
            bj = extract(sb, j)
            sj = extract(ss, j)
            pltpu.async_copy(st_hbm.at[pl.ds(bj, 1), pl.ds(sj, 1)],
                             s_v.at[pl.ds(j, 1)], sem)
            return 0

        lax.fori_loop(0, _L, fire_g, 0, unroll=4)
        lax.fori_loop(0, _L, fire_s, 0, unroll=4)
        for t in range(_NEG):
            nb, ns = nbs[t]

            def fire_n(j, _, nb=nb, ns=ns, t=t):
                bj = extract(nb, j)
                sj = extract(ns, j)
                pltpu.async_copy(st_hbm.at[pl.ds(bj, 1), pl.ds(sj, 1)],
                                 n_v.at[pl.ds(t * _L + j, 1)], sem)
                return 0

            lax.fori_loop(0, _L, fire_n, 0, unroll=4)

    def drain(buf):
        g_v, _, _, sem = buf

        def drain_one(j, _):
            pltpu.make_async_copy(
                gt_hbm.at[pl.ds(0, 1), pl.ds(0, 1)],
                g_v.at[pl.ds(0, 1)], sem).wait()
            return 0

        lax.fori_loop(0, (2 + _NEG) * _L, drain_one, 0, unroll=4)

    def compute(c, buf):
        g_v, s_v, n_v, _ = buf
        coff = pl.multiple_of(c * _CHUNK, _CHUNK)

        def body(d, accs):
            dd = jnp.full((_L,), d, jnp.int32)
            gcol = plsc.load_gather(g_v, [iota, zeros, dd])
            scol = plsc.load_gather(s_v, [iota, zeros, dd])
            out = [accs[0] + gcol * scol]
            for k in range(_NEG):
                ncol = plsc.load_gather(n_v, [iota * _NEG + k, zeros, dd])
                out.append(accs[k + 1] + gcol * ncol)
            return tuple(out)

        z = jnp.zeros((_L,), jnp.float32)
        accs = lax.fori_loop(0, _D, body, (z,) * (1 + _NEG), unroll=4)
        pos_v[pl.ds(coff, _L)] = accs[0]
        for k in range(_NEG):
            neg_v[pl.ds(pl.multiple_of(k * _PER_W + coff, _L), _L)] = (
                accs[k + 1])

    fire(0, bufs[0])

    def pair_body(p, carry):
        c0 = p * 2
        fire(c0 + 1, bufs[1])
        drain(bufs[0])
        compute(c0, bufs[0])

        @pl.when(p < _NCHUNK // 2 - 1)
        def _():
            fire(c0 + 2, bufs[0])

        drain(bufs[1])
        compute(c0 + 1, bufs[1])
        return carry

    lax.fori_loop(0, _NCHUNK // 2, pair_body, 0)
    pltpu.sync_copy(pos_v, pos_hbm.at[pl.ds(wbase, _PER_W)])
    for k in range(_NEG):
        pltpu.sync_copy(neg_v.at[pl.ds(k * _PER_W, _PER_W)],
                        neg_hbm.at[pl.ds(k * _B + wbase, _PER_W)])


_sc_scores = pl.kernel(
    _sc_scores_body,
    out_type=[jax.ShapeDtypeStruct((_B,), jnp.float32),
              jax.ShapeDtypeStruct((_NEG * _B,), jnp.float32)],
    mesh=plsc.VectorSubcoreMesh(core_axis_name="c", subcore_axis_name="s",
                                num_cores=_NC, num_subcores=_NS),
    scratch_types=[
        pltpu.VMEM((_PER_W,), jnp.int32),
        pltpu.VMEM((_PER_W,), jnp.int32),
        pltpu.VMEM((_PER_W * _NEG,), jnp.int32),
        pltpu.VMEM((_L, 1, _D), jnp.float32),
        pltpu.VMEM((_L, 1, _D), jnp.float32),
        pltpu.VMEM((_NKC, 1, _D), jnp.float32),
        pltpu.VMEM((_L, 1, _D), jnp.float32),
        pltpu.VMEM((_L, 1, _D), jnp.float32),
        pltpu.VMEM((_NKC, 1, _D), jnp.float32),
        pltpu.VMEM((_PER_W,), jnp.float32),
        pltpu.VMEM((_NEG * _PER_W,), jnp.float32),
        pltpu.SemaphoreType.DMA,
        pltpu.SemaphoreType.DMA,
    ],
    compiler_params=pltpu.CompilerParams(needs_layout_passes=False,
                                         use_tc_tiling_on_sc=True),
)


def _tc_loss_body(pos_ref, neg_ref, out_ref):
    p = pos_ref[...]
    pos_loss = -jnp.log(jax.nn.sigmoid(p) + 1e-8)
    acc = jnp.zeros_like(p)
    for k in range(_NEG):
        acc = acc + (-jnp.log(1.0 - jax.nn.sigmoid(neg_ref[k]) + 1e-8))
    out_ref[...] = pos_loss + acc * (1.0 / _NEG)


@jax.jit
def _impl(graph_idx, subgraph_idx, neg_idx, graph_table, subgraph_table):
    nidx_flat = neg_idx.reshape(-1)
    gt3 = graph_table.reshape(-1, _SL, _D)
    st3 = subgraph_table.reshape(-1, _SL, _D)
    pos, negf = _sc_scores(gt3, st3, graph_idx, subgraph_idx, nidx_flat)
    r = _B // 128
    loss = pl.pallas_call(
        _tc_loss_body,
        out_shape=jax.ShapeDtypeStruct((r, 128), jnp.float32),
    )(pos.reshape(r, 128), negf.reshape(_NEG, r, 128))
    return loss.reshape(_B)


def kernel(graph_idx, subgraph_idx, neg_idx, graph_table, subgraph_table):
    return _impl(graph_idx, subgraph_idx, neg_idx, graph_table,
                 subgraph_table)
